# baseline (speedup 1.0000x reference)
import functools

import jax
import jax.numpy as jnp
from jax.experimental import pallas as pl
from jax.experimental.pallas import tpu as pltpu


def _pos_kernel(row_t_ref, col_t_ref, out_ref, scratch_ref, *, h, w, d):
    @pl.when(pl.program_id(0) == 0)
    def _():
        col_t = col_t_ref[:, :w]
        row_t = row_t_ref[:, :h]
        x_part = jnp.broadcast_to(col_t[:, None, :], (d, h, w))
        y_part = jnp.broadcast_to(row_t[:, :, None], (d, h, w))
        scratch_ref[:d] = x_part
        scratch_ref[d:] = y_part

    out_ref[...] = scratch_ref[...][None]


def kernel(inputs, row_embed, col_embed):
    h, w = inputs.shape[-2], inputs.shape[-1]
    b = inputs.shape[0]
    d = row_embed.shape[1]

    row_t = row_embed.T
    col_t = col_embed.T

    out = pl.pallas_call(
        functools.partial(_pos_kernel, h=h, w=w, d=d),
        grid=(b,),
        in_specs=[
            pl.BlockSpec(row_t.shape, lambda i: (0, 0)),
            pl.BlockSpec(col_t.shape, lambda i: (0, 0)),
        ],
        out_specs=pl.BlockSpec((1, 2 * d, h, w), lambda i: (i, 0, 0, 0)),
        out_shape=jax.ShapeDtypeStruct((b, 2 * d, h, w), jnp.float32),
        scratch_shapes=[pltpu.VMEM((2 * d, h, w), jnp.float32)],
    )(row_t, col_t)
    return out


# speedup vs baseline: 2.6688x; 2.6688x over previous
import functools

import jax
import jax.numpy as jnp
from jax import lax
from jax.experimental import pallas as pl
from jax.experimental.pallas import tpu as pltpu


def _pos_kernel(row_ref, col_ref, out_ref, scratch_ref, *, h, w, d):
    @pl.when(pl.program_id(0) == 0)
    def _():
        hw = h * w
        lane_x = lax.broadcasted_iota(jnp.int32, (w, hw), 1)
        sub_x = lax.broadcasted_iota(jnp.int32, (w, hw), 0)
        s_x = (lane_x % w == sub_x).astype(jnp.float32)
        lane_y = lax.broadcasted_iota(jnp.int32, (h, hw), 1)
        sub_y = lax.broadcasted_iota(jnp.int32, (h, hw), 0)
        s_y = (lane_y // w == sub_y).astype(jnp.float32)
        dn = (((0,), (0,)), ((), ()))
        col = col_ref[:w, :]
        row = row_ref[:h, :]
        x_part = lax.dot_general(col, s_x, dn, preferred_element_type=jnp.float32)
        y_part = lax.dot_general(row, s_y, dn, preferred_element_type=jnp.float32)
        scratch_ref[:d] = x_part
        scratch_ref[d:] = y_part
    out_ref[...] = scratch_ref[...][None]


def kernel(inputs, row_embed, col_embed):
    h, w = inputs.shape[-2], inputs.shape[-1]
    b = inputs.shape[0]
    d = row_embed.shape[1]
    hw = h * w

    out = pl.pallas_call(
        functools.partial(_pos_kernel, h=h, w=w, d=d),
        grid=(b,),
        in_specs=[
            pl.BlockSpec(row_embed.shape, lambda i: (0, 0)),
            pl.BlockSpec(col_embed.shape, lambda i: (0, 0)),
        ],
        out_specs=pl.BlockSpec((1, 2 * d, hw), lambda i: (i, 0, 0)),
        out_shape=jax.ShapeDtypeStruct((b, 2 * d, hw), jnp.float32),
        scratch_shapes=[pltpu.VMEM((2 * d, hw), jnp.float32)],
    )(row_embed, col_embed)
    return out.reshape(b, 2 * d, h, w)


# speedup vs baseline: 3.1373x; 1.1755x over previous
import functools

import jax
import jax.numpy as jnp
from jax import lax
from jax.experimental import pallas as pl
from jax.experimental.pallas import tpu as pltpu


def _pos_kernel(row_ref, col_ref, out_ref, scratch_ref, sem, *, b, h, w, d):
    hw = h * w
    lane_x = lax.broadcasted_iota(jnp.int32, (w, hw), 1)
    sub_x = lax.broadcasted_iota(jnp.int32, (w, hw), 0)
    s_x = (lane_x % w == sub_x).astype(jnp.float32)
    lane_y = lax.broadcasted_iota(jnp.int32, (h, hw), 1)
    sub_y = lax.broadcasted_iota(jnp.int32, (h, hw), 0)
    s_y = (lane_y // w == sub_y).astype(jnp.float32)
    dn = (((0,), (0,)), ((), ()))
    col = col_ref[:w, :]
    row = row_ref[:h, :]
    x_part = lax.dot_general(col, s_x, dn, preferred_element_type=jnp.float32)
    y_part = lax.dot_general(row, s_y, dn, preferred_element_type=jnp.float32)
    scratch_ref[:d] = x_part
    scratch_ref[d:] = y_part

    copies = [
        pltpu.make_async_copy(scratch_ref, out_ref.at[i], sem) for i in range(b)
    ]
    for c in copies:
        c.start()
    for c in copies:
        c.wait()


def kernel(inputs, row_embed, col_embed):
    h, w = inputs.shape[-2], inputs.shape[-1]
    b = inputs.shape[0]
    d = row_embed.shape[1]
    hw = h * w

    out = pl.pallas_call(
        functools.partial(_pos_kernel, b=b, h=h, w=w, d=d),
        in_specs=[
            pl.BlockSpec(row_embed.shape, lambda: (0, 0)),
            pl.BlockSpec(col_embed.shape, lambda: (0, 0)),
        ],
        out_specs=pl.BlockSpec(memory_space=pl.ANY),
        out_shape=jax.ShapeDtypeStruct((b, 2 * d, hw), jnp.float32),
        scratch_shapes=[
            pltpu.VMEM((2 * d, hw), jnp.float32),
            pltpu.SemaphoreType.DMA,
        ],
    )(row_embed, col_embed)
    return out.reshape(b, 2 * d, h, w)
